# baseline (speedup 1.0000x reference)
import functools
import math

import jax
import jax.numpy as jnp
from jax import lax
from jax.experimental import pallas as pl
from jax.experimental.pallas import tpu as pltpu
from jax.experimental.pallas import tpu_sc as plsc

B, T, D, G = 16, 4096, 128, 128
L = 16
NV = D // L
TW = (B * T) // 32
CT = 256
NCH = TW // CT
EPS = float(jnp.finfo(jnp.float32).eps)
INV_D = 1.0 / D
UNROLL = 16


def _rsqrt_vec(m):
    i = lax.bitcast_convert_type(m, jnp.int32)
    i = jnp.full((L,), 0x5F3759DF, jnp.int32) - lax.shift_right_arithmetic(
        i, jnp.full((L,), 1, jnp.int32))
    y = lax.bitcast_convert_type(i, jnp.float32)
    hm = 0.5 * m
    for _ in range(2):
        y = y * (1.5 - hm * y * y)
    return y


def _allsum(v, perms):
    dnums = lax.GatherDimensionNumbers(
        offset_dims=(), collapsed_slice_dims=(0,), start_index_map=(0,))
    for p in perms:
        v = v + lax.gather(v, p[:, None], dnums, slice_sizes=(1,),
                           mode=lax.GatherScatterMode.PROMISE_IN_BOUNDS)
    return v


def _sc_pool(x_hbm, gid_hbm, qw_hbm, acc_hbm, den_hbm,
             xbuf, gbuf, qwv, accv, denv, sx0, sx1, sg0, sg1):
    b = lax.axis_index("s")
    h = lax.axis_index("c")
    t_base = h * TW + b * 0

    pltpu.sync_copy(qw_hbm, qwv)
    qw = [qwv[pl.ds(L * j, L)] for j in range(NV)]

    iota = lax.iota(jnp.int32, L)
    perms = [lax.rem(iota + (1 << k), jnp.full((L,), L, jnp.int32))
             for k in (3, 2, 1, 0)]

    sx = [sx0, sx1]
    sg = [sg0, sg1]

    def start(i):
        slot = i % 2
        t0 = t_base + i * CT
        hx = pltpu.async_copy(x_hbm.at[b, pl.ds(t0, CT), :], xbuf.at[slot],
                              sx[slot])
        hg = pltpu.async_copy(gid_hbm.at[b, pl.ds(t0, CT)], gbuf.at[slot],
                              sg[slot])
        return hx, hg

    pending = [None, None]
    pending[0] = start(0)

    zv = jnp.zeros((L,), jnp.float32)

    def zbody(i, c):
        for j in range(NV):
            accv[i, pl.ds(L * j, L)] = zv
        denv[i, :] = zv
        return c

    lax.fori_loop(0, G, zbody, 0)

    def chunk_body(slot, carry):
        def blk_body(it, carry):
            cum, den_v = carry
            gv = gbuf[slot, pl.ds(it * UNROLL, UNROLL)]
            for u in range(UNROLL):
                t = it * UNROLL + u
                g = gv[u]
                xs = [xbuf[slot, t, pl.ds(L * j, L)] for j in range(NV)]
                psum = xs[0] * qw[0]
                qsum = xs[0] * xs[0]
                for j in range(1, NV):
                    psum = psum + xs[j] * qw[j]
                    qsum = qsum + xs[j] * xs[j]
                s_v = _allsum(psum, perms)
                ssq_v = _allsum(qsum, perms)
                r_v = _rsqrt_vec(ssq_v * INV_D + EPS)
                e_v = jnp.exp(s_v * r_v)
                a_v = e_v * r_v
                den_v = den_v + e_v
                cum = [cum[j] + a_v * xs[j] for j in range(NV)]
                for j in range(NV):
                    accv[g, pl.ds(L * j, L)] = cum[j]
                denv[g, :] = den_v
            return cum, den_v

        return lax.fori_loop(0, CT // UNROLL, blk_body, carry)

    carry = ([zv] * NV, zv)
    for i in range(NCH):
        if i + 1 < NCH:
            pending[(i + 1) % 2] = start(i + 1)
        hx, hg = pending[i % 2]
        hx.wait()
        hg.wait()
        carry = chunk_body(i % 2, carry)

    def post_body(g, fills):
        fill_d, fill_c = fills
        raw_d = denv[g, :]
        present = raw_d != 0.0
        denv[g, :] = jnp.where(present, raw_d - fill_d, 0.0)
        new_fill_d = jnp.where(present, raw_d, fill_d)
        new_fill_c = []
        for j in range(NV):
            raw = accv[g, pl.ds(L * j, L)]
            accv[g, pl.ds(L * j, L)] = jnp.where(present, raw - fill_c[j], 0.0)
            new_fill_c.append(jnp.where(present, raw, fill_c[j]))
        return new_fill_d, new_fill_c

    lax.fori_loop(0, G, post_body, (zv, [zv] * NV))

    pltpu.sync_copy(accv, acc_hbm.at[b, h])
    pltpu.sync_copy(denv, den_hbm.at[b, h])


def _qw_body(q_ref, wk_ref, nw_ref, qw_ref):
    qk = jnp.dot(q_ref[...], wk_ref[...], preferred_element_type=jnp.float32)
    qw_ref[...] = qk * nw_ref[...] * (1.0 / math.sqrt(D))


def _merge_body(acc_ref, den_ref, nw_ref, wv_ref, out_ref):
    b = pl.program_id(0)
    A = acc_ref[0, 0] + acc_ref[0, 1]
    dpair = den_ref[b]
    d = dpair[0] + dpair[1]
    inv = jnp.where(d > 0, 1.0 / d, 0.0)
    M = A * nw_ref[...] * inv[:, None]
    out_ref[0] = lax.dot_general(M, wv_ref[...], (((1,), (1,)), ((), ())),
                                 preferred_element_type=jnp.float32)


@jax.jit
def _run(x, group_id, query, norm_w, Wk, Wv):
    qw2 = pl.pallas_call(
        _qw_body,
        out_shape=jax.ShapeDtypeStruct((1, D), jnp.float32),
    )(query.reshape(1, D), Wk, norm_w.reshape(1, D))
    qw = qw2.reshape(D)

    mesh = plsc.VectorSubcoreMesh(core_axis_name="c", subcore_axis_name="s")
    sc = functools.partial(
        pl.kernel,
        mesh=mesh,
        out_type=[
            jax.ShapeDtypeStruct((B, 2, G, D), jnp.float32),
            jax.ShapeDtypeStruct((B, 2, G, L), jnp.float32),
        ],
        scratch_types=[
            pltpu.VMEM((2, CT, D), jnp.float32),
            pltpu.VMEM((2, CT), jnp.int32),
            pltpu.VMEM((D,), jnp.float32),
            pltpu.VMEM((G, D), jnp.float32),
            pltpu.VMEM((G, L), jnp.float32),
            pltpu.SemaphoreType.DMA,
            pltpu.SemaphoreType.DMA,
            pltpu.SemaphoreType.DMA,
            pltpu.SemaphoreType.DMA,
        ],
    )(_sc_pool)
    acc, den4 = sc(x, group_id.astype(jnp.int32), qw)
    den = den4[..., 0]

    out = pl.pallas_call(
        _merge_body,
        grid=(B,),
        in_specs=[
            pl.BlockSpec((1, 2, G, D), lambda b: (b, 0, 0, 0)),
            pl.BlockSpec((B, 2, G), lambda b: (0, 0, 0)),
            pl.BlockSpec((1, D), lambda b: (0, 0)),
            pl.BlockSpec((D, D), lambda b: (0, 0)),
        ],
        out_specs=pl.BlockSpec((1, G, D), lambda b: (b, 0, 0)),
        out_shape=jax.ShapeDtypeStruct((B, G, D), jnp.float32),
    )(acc, den, norm_w.reshape(1, D), Wv)
    return out


def kernel(x, group_id, num_groups, query, norm_w, Wk, Wv):
    return _run(x, group_id, query, norm_w, Wk, Wv)


# speedup vs baseline: 1.3407x; 1.3407x over previous
import functools
import math

import jax
import jax.numpy as jnp
from jax import lax
from jax.experimental import pallas as pl
from jax.experimental.pallas import tpu as pltpu
from jax.experimental.pallas import tpu_sc as plsc

B, T, D, G = 16, 4096, 128, 128
L = 16
NV = D // L
TW = (B * T) // 32
CT = 64
NCH = TW // CT
EPS = float(jnp.finfo(jnp.float32).eps)
UNROLL = 16


def _score_body(x_ref, q_ref, nw_ref, wk_ref, a_ref, e_ref):
    qk = jnp.dot(q_ref[...], wk_ref[...], preferred_element_type=jnp.float32)
    qw = qk * nw_ref[...] * (1.0 / math.sqrt(D))
    xb = x_ref[0]
    ssq = jnp.sum(xb * xb, axis=1, keepdims=True)
    r = lax.rsqrt(ssq * (1.0 / D) + EPS)
    s = lax.dot_general(xb, qw, (((1,), (1,)), ((), ())),
                        preferred_element_type=jnp.float32)
    e = jnp.exp(s * r)
    a = e * r
    a_ref[0] = jnp.broadcast_to(a, (T, L))
    e_ref[0] = jnp.broadcast_to(e, (T, L))


def _sc_pool(x_hbm, gid_hbm, a_hbm, e_hbm, acc_hbm, den_hbm,
             xbuf, gbuf, abuf, ebuf, accv, denv,
             sx0, sx1, sg0, sg1, sa0, sa1, se0, se1):
    b = lax.axis_index("s")
    h = lax.axis_index("c")
    t_base = h * TW

    sx = [sx0, sx1]
    sg = [sg0, sg1]
    sa = [sa0, sa1]
    se = [se0, se1]

    def copies(i, slot):
        t0 = t_base + i * CT
        return [
            pltpu.make_async_copy(x_hbm.at[b, pl.ds(t0, CT), :],
                                  xbuf.at[slot], sx[slot]),
            pltpu.make_async_copy(gid_hbm.at[b, pl.ds(t0, CT)],
                                  gbuf.at[slot], sg[slot]),
            pltpu.make_async_copy(a_hbm.at[b, pl.ds(t0, CT), :],
                                  abuf.at[slot], sa[slot]),
            pltpu.make_async_copy(e_hbm.at[b, pl.ds(t0, CT), :],
                                  ebuf.at[slot], se[slot]),
        ]

    def start(i, slot):
        for c in copies(i, slot):
            c.start()

    def wait(i, slot):
        for c in copies(i, slot):
            c.wait()

    start(0, 0)
    start(1, 1)

    zv = jnp.zeros((L,), jnp.float32)

    def zbody(i, c):
        for j in range(NV):
            accv[i, pl.ds(L * j, L)] = zv
        denv[i, :] = zv
        return c

    lax.fori_loop(0, G, zbody, 0)

    def chunk_body(slot, carry):
        def blk_body(it, carry):
            cum, den_v = carry
            gv = gbuf[slot, pl.ds(it * UNROLL, UNROLL)]
            for u in range(UNROLL):
                t = it * UNROLL + u
                a_bc = abuf[slot, t, :]
                e_bc = ebuf[slot, t, :]
                xs = [xbuf[slot, t, pl.ds(L * j, L)] for j in range(NV)]
                den_v = den_v + e_bc
                cum = [cum[j] + a_bc * xs[j] for j in range(NV)]
                for j in range(NV):
                    accv[gv[u], pl.ds(L * j, L)] = cum[j]
                denv[gv[u], :] = den_v
            return cum, den_v

        return lax.fori_loop(0, CT // UNROLL, blk_body, carry)

    def pair_body(p, carry):
        for k in range(2):
            i = 2 * p + k
            wait(i, k)
            carry = chunk_body(k, carry)
            start(i + 2, k)
        return carry

    carry = lax.fori_loop(0, NCH // 2 - 1, pair_body, ([zv] * NV, zv))
    for k in range(2):
        wait(NCH - 2 + k, k)
        carry = chunk_body(k, carry)

    def post_body(g, fills):
        fill_d, fill_c = fills
        raw_d = denv[g, :]
        present = raw_d != 0.0
        denv[g, :] = jnp.where(present, raw_d - fill_d, 0.0)
        new_fill_d = jnp.where(present, raw_d, fill_d)
        new_fill_c = []
        for j in range(NV):
            raw = accv[g, pl.ds(L * j, L)]
            accv[g, pl.ds(L * j, L)] = jnp.where(present, raw - fill_c[j],
                                                 0.0)
            new_fill_c.append(jnp.where(present, raw, fill_c[j]))
        return new_fill_d, new_fill_c

    lax.fori_loop(0, G, post_body, (zv, [zv] * NV))

    pltpu.sync_copy(accv, acc_hbm.at[b, h])
    pltpu.sync_copy(denv, den_hbm.at[b, h])


def _merge_body(acc_ref, den_ref, nw_ref, wv_ref, out_ref):
    b = pl.program_id(0)
    A = acc_ref[0, 0] + acc_ref[0, 1]
    dpair = den_ref[b]
    d = dpair[0] + dpair[1]
    inv = jnp.where(d > 0, 1.0 / d, 0.0)
    M = A * nw_ref[...] * inv[:, None]
    out_ref[0] = lax.dot_general(M, wv_ref[...], (((1,), (1,)), ((), ())),
                                 preferred_element_type=jnp.float32)


@jax.jit
def _run(x, group_id, query, norm_w, Wk, Wv):
    abuf, ebuf = pl.pallas_call(
        _score_body,
        grid=(B,),
        in_specs=[
            pl.BlockSpec((1, T, D), lambda b: (b, 0, 0)),
            pl.BlockSpec((1, D), lambda b: (0, 0)),
            pl.BlockSpec((1, D), lambda b: (0, 0)),
            pl.BlockSpec((D, D), lambda b: (0, 0)),
        ],
        out_specs=[
            pl.BlockSpec((1, T, L), lambda b: (b, 0, 0)),
            pl.BlockSpec((1, T, L), lambda b: (b, 0, 0)),
        ],
        out_shape=[
            jax.ShapeDtypeStruct((B, T, L), jnp.float32),
            jax.ShapeDtypeStruct((B, T, L), jnp.float32),
        ],
    )(x, query.reshape(1, D), norm_w.reshape(1, D), Wk)

    mesh = plsc.VectorSubcoreMesh(core_axis_name="c", subcore_axis_name="s")
    sc = functools.partial(
        pl.kernel,
        mesh=mesh,
        out_type=[
            jax.ShapeDtypeStruct((B, 2, G, D), jnp.float32),
            jax.ShapeDtypeStruct((B, 2, G, L), jnp.float32),
        ],
        scratch_types=[
            pltpu.VMEM((2, CT, D), jnp.float32),
            pltpu.VMEM((2, CT), jnp.int32),
            pltpu.VMEM((2, CT, L), jnp.float32),
            pltpu.VMEM((2, CT, L), jnp.float32),
            pltpu.VMEM((G, D), jnp.float32),
            pltpu.VMEM((G, L), jnp.float32),
            pltpu.SemaphoreType.DMA,
            pltpu.SemaphoreType.DMA,
            pltpu.SemaphoreType.DMA,
            pltpu.SemaphoreType.DMA,
            pltpu.SemaphoreType.DMA,
            pltpu.SemaphoreType.DMA,
            pltpu.SemaphoreType.DMA,
            pltpu.SemaphoreType.DMA,
        ],
    )(_sc_pool)
    acc, den4 = sc(x, group_id.astype(jnp.int32), abuf, ebuf)
    den = den4[..., 0]

    out = pl.pallas_call(
        _merge_body,
        grid=(B,),
        in_specs=[
            pl.BlockSpec((1, 2, G, D), lambda b: (b, 0, 0, 0)),
            pl.BlockSpec((B, 2, G), lambda b: (0, 0, 0)),
            pl.BlockSpec((1, D), lambda b: (0, 0)),
            pl.BlockSpec((D, D), lambda b: (0, 0)),
        ],
        out_specs=pl.BlockSpec((1, G, D), lambda b: (b, 0, 0)),
        out_shape=jax.ShapeDtypeStruct((B, G, D), jnp.float32),
    )(acc, den, norm_w.reshape(1, D), Wv)
    return out


def kernel(x, group_id, num_groups, query, norm_w, Wk, Wv):
    return _run(x, group_id, query, norm_w, Wk, Wv)
